# Initial kernel scaffold; baseline (speedup 1.0000x reference)
import jax
import jax.numpy as jnp
from jax.experimental import pallas as pl


def kernel(x, table):
    raise NotImplementedError("write your pallas kernel here")


# speedup vs baseline: 1.6839x; 1.6839x over previous
import functools

import jax
import jax.numpy as jnp
from jax import lax
from jax.experimental import pallas as pl
from jax.experimental.pallas import tpu as pltpu
from jax.experimental.pallas import tpu_sc as plsc

_NC = 2
_NS = 16
_NW = _NC * _NS
_CHUNK = 128


@functools.lru_cache(maxsize=None)
def _make_kernel(n_idx, d_embed):
    assert n_idx % (_NW * _CHUNK) == 0
    per_tile = n_idx // _NW
    nchunk = per_tile // _CHUNK
    mesh = plsc.VectorSubcoreMesh(core_axis_name="c", subcore_axis_name="s")

    @functools.partial(
        pl.kernel,
        mesh=mesh,
        out_type=jax.ShapeDtypeStruct((n_idx, d_embed), jnp.float32),
        scratch_types=[
            pltpu.VMEM((nchunk, _CHUNK), jnp.int32),
            pltpu.VMEM((_CHUNK, d_embed), jnp.float32),
            pltpu.SemaphoreType.DMA,
        ],
        compiler_params=pltpu.CompilerParams(use_tc_tiling_on_sc=False),
    )
    def gather_kernel(idx_hbm, table_hbm, out_hbm, idx_v, rows, gsem):
        wid = lax.axis_index("s") * _NC + lax.axis_index("c")
        base = wid * per_tile
        pltpu.sync_copy(idx_hbm.at[wid], idx_v)

        def chunk_body(j, carry):
            pltpu.async_copy(table_hbm.at[idx_v.at[j]], rows, gsem).wait()
            pltpu.sync_copy(rows, out_hbm.at[pl.ds(base + j * _CHUNK, _CHUNK)])
            return carry

        lax.fori_loop(0, nchunk, chunk_body, 0)

    return gather_kernel


def kernel(x, table):
    batch, hist = x.shape
    n_idx = batch * hist
    d_embed = table.shape[1]
    idx = x.astype(jnp.int32).reshape(_NW, n_idx // (_NW * _CHUNK), _CHUNK)
    out = _make_kernel(n_idx, d_embed)(idx, table)
    return out.reshape(batch, hist, d_embed)


# speedup vs baseline: 1.8770x; 1.1147x over previous
import functools

import jax
import jax.numpy as jnp
from jax import lax
from jax.experimental import pallas as pl
from jax.experimental.pallas import tpu as pltpu
from jax.experimental.pallas import tpu_sc as plsc

_NC = 2
_NS = 16
_NW = _NC * _NS
_CHUNK = 128


_NBUF = 4
_GROUP = _NBUF * _CHUNK


@functools.lru_cache(maxsize=None)
def _make_kernel(n_idx, d_embed):
    assert n_idx % (_NW * 2 * _GROUP) == 0
    per_tile = n_idx // _NW
    nchunk = per_tile // _CHUNK
    ngroups = nchunk // _NBUF
    npairs = ngroups // 2
    mesh = plsc.VectorSubcoreMesh(core_axis_name="c", subcore_axis_name="s")

    @functools.partial(
        pl.kernel,
        mesh=mesh,
        out_type=jax.ShapeDtypeStruct((n_idx, d_embed), jnp.float32),
        scratch_types=[
            pltpu.VMEM((nchunk, _CHUNK), jnp.int32),
            pltpu.VMEM((_GROUP, d_embed), jnp.float32),
            pltpu.VMEM((_GROUP, d_embed), jnp.float32),
            pltpu.SemaphoreType.DMA,
            pltpu.SemaphoreType.DMA,
            pltpu.SemaphoreType.DMA,
            pltpu.SemaphoreType.DMA,
        ],
        compiler_params=pltpu.CompilerParams(use_tc_tiling_on_sc=False),
    )
    def gather_kernel(idx_hbm, table_hbm, out_hbm, idx_v, buf_a, buf_b,
                      gsem_a, gsem_b, ssem_a, ssem_b):
        wid = lax.axis_index("s") * _NC + lax.axis_index("c")
        base = wid * per_tile
        pltpu.sync_copy(idx_hbm.at[wid], idx_v)

        def fire(buf, gsem, g):
            for b in range(_NBUF):
                pltpu.async_copy(
                    table_hbm.at[idx_v.at[g * _NBUF + b]],
                    buf.at[pl.ds(b * _CHUNK, _CHUNK)],
                    gsem,
                )

        def phase(buf, gsem, ssem, g, refill_g, do_refill):
            pltpu.make_async_copy(out_hbm.at[pl.ds(0, _GROUP)], buf, gsem).wait()
            sc = pltpu.async_copy(
                buf, out_hbm.at[pl.ds(base + g * _GROUP, _GROUP)], ssem)
            sc.wait()
            if do_refill:
                fire(buf, gsem, refill_g)

        fire(buf_a, gsem_a, 0)
        fire(buf_b, gsem_b, 1)

        def pair_body(t, carry):
            g = 2 * t
            phase(buf_a, gsem_a, ssem_a, g, g + 2, True)
            phase(buf_b, gsem_b, ssem_b, g + 1, g + 3, True)
            return carry

        lax.fori_loop(0, npairs - 1, pair_body, 0)
        g_last = 2 * (npairs - 1)
        phase(buf_a, gsem_a, ssem_a, g_last, 0, False)
        phase(buf_b, gsem_b, ssem_b, g_last + 1, 0, False)

    return gather_kernel


def kernel(x, table):
    batch, hist = x.shape
    n_idx = batch * hist
    d_embed = table.shape[1]
    idx = x.astype(jnp.int32).reshape(_NW, n_idx // (_NW * _CHUNK), _CHUNK)
    out = _make_kernel(n_idx, d_embed)(idx, table)
    return out.reshape(batch, hist, d_embed)
